# Initial kernel scaffold; baseline (speedup 1.0000x reference)
import functools

import jax
import jax.numpy as jnp
from jax import lax
from jax.experimental import pallas as pl
from jax.experimental.pallas import tpu as pltpu
from jax.experimental.pallas import tpu_sc as plsc

N = 10000
H = 128
OUT = 16
R = 3
E = 200000

NC = 2
NS = 16
NW = NC * NS
CHUNK = 128

NP = 10240
ET = R * E
NCHUNK = -(-ET // (NW * CHUNK))
EPW = NCHUNK * CHUNK
ETP = NW * EPW
RPS = NP // NS


def _make_seg_pass(tab_rows, d):
    mesh = plsc.VectorSubcoreMesh(core_axis_name="c", subcore_axis_name="s")

    @functools.partial(
        pl.kernel,
        out_type=jax.ShapeDtypeStruct((NC * NP, d), jnp.float32),
        mesh=mesh,
        scratch_types=[
            pltpu.VMEM((CHUNK,), jnp.int32),
            pltpu.VMEM((CHUNK,), jnp.int32),
            pltpu.VMEM((CHUNK, d), jnp.float32),
            pltpu.VMEM_SHARED((NP, d), jnp.float32),
            pltpu.SemaphoreType.DMA,
        ],
    )
    def seg_pass(tab, src, dst, out, src_v, dst_v, rows_v, acc, sem):
        cid = lax.axis_index("c")
        sid = lax.axis_index("s")
        wid = sid * NC + cid

        zeros16 = jnp.zeros((16,), jnp.float32)

        def zstore(i, carry):
            for k in range(d // 16):
                rows_v[i, pl.ds(k * 16, 16)] = zeros16
            return carry

        lax.fori_loop(0, CHUNK, zstore, 0)

        def zcopy(i, carry):
            pltpu.sync_copy(
                rows_v, acc.at[pl.ds(sid * RPS + i * CHUNK, CHUNK)])
            return carry

        lax.fori_loop(0, RPS // CHUNK, zcopy, 0)
        plsc.subcore_barrier()

        def body(i, carry):
            base = wid * EPW + i * CHUNK
            pltpu.sync_copy(src.at[pl.ds(base, CHUNK)], src_v)
            pltpu.sync_copy(dst.at[pl.ds(base, CHUNK)], dst_v)
            pltpu.async_copy(tab.at[src_v], rows_v, sem).wait()
            pltpu.sync_copy(rows_v, acc.at[dst_v], add=True)
            return carry

        lax.fori_loop(0, NCHUNK, body, 0)
        plsc.subcore_barrier()

        pltpu.sync_copy(
            acc.at[pl.ds(sid * RPS, RPS)],
            out.at[pl.ds(cid * NP + sid * RPS, RPS)])

    return seg_pass


def _combine_mm_body(p_ref, b_ref, w_ref, y_ref):
    h = jnp.maximum(p_ref[0] + p_ref[1] + b_ref[0][None, :], 0.0)
    for r in range(R):
        y_ref[r] = jnp.dot(h, w_ref[r], preferred_element_type=jnp.float32)


def _combine_mm(part, b, w, d2):
    bn = 512
    return pl.pallas_call(
        _combine_mm_body,
        grid=(NP // bn,),
        in_specs=[
            pl.BlockSpec((NC, bn, H), lambda i: (0, i, 0)),
            pl.BlockSpec((1, H), lambda i: (0, 0)),
            pl.BlockSpec((R, H, d2), lambda i: (0, 0, 0)),
        ],
        out_specs=pl.BlockSpec((R, bn, d2), lambda i: (0, i, 0)),
        out_shape=jax.ShapeDtypeStruct((R, NP, d2), jnp.float32),
    )(part, b.reshape(1, H), w)


def _final_body(p_ref, b_ref, o_ref):
    o_ref[...] = p_ref[0] + p_ref[1] + b_ref[0][None, :]


def _final(part, b):
    bn = 512
    return pl.pallas_call(
        _final_body,
        grid=(NP // bn,),
        in_specs=[
            pl.BlockSpec((NC, bn, OUT), lambda i: (0, i, 0)),
            pl.BlockSpec((1, OUT), lambda i: (0, 0)),
        ],
        out_specs=pl.BlockSpec((bn, OUT), lambda i: (i, 0)),
        out_shape=jax.ShapeDtypeStruct((NP, OUT), jnp.float32),
    )(part, b.reshape(1, OUT))


def kernel(embed, b0, w1, b1, w2, b2, edge_index_0, edge_index_1,
           edge_index_2):
    ei = [edge_index_0, edge_index_1, edge_index_2]
    pad = ETP - ET
    pad_src = jnp.zeros((pad,), jnp.int32)
    pad_dst = jnp.full((pad,), N, jnp.int32)

    src_a = jnp.concatenate([e[0] for e in ei] + [pad_src])
    src_b = jnp.concatenate(
        [ei[r][0] + r * NP for r in range(R)] + [pad_src])
    dst_p = jnp.concatenate([e[1] for e in ei] + [pad_dst])

    part_a = _make_seg_pass(N, H)(embed, src_a, dst_p)
    y = _combine_mm(part_a.reshape(NC, NP, H), b0, w1, H)

    part_b = _make_seg_pass(R * NP, H)(y.reshape(R * NP, H), src_b, dst_p)
    z = _combine_mm(part_b.reshape(NC, NP, H), b1, w2, OUT)

    part_c = _make_seg_pass(R * NP, OUT)(
        z.reshape(R * NP, OUT), src_b, dst_p)
    out = _final(part_c.reshape(NC, NP, OUT), b2)
    return out[:N]


# speedup vs baseline: 3.8407x; 3.8407x over previous
import functools

import jax
import jax.numpy as jnp
from jax import lax
from jax.experimental import pallas as pl
from jax.experimental.pallas import tpu as pltpu
from jax.experimental.pallas import tpu_sc as plsc

N = 10000
H = 128
OUT = 16
R = 3
E = 200000

NC = 2
NS = 16
NW = NC * NS
CHUNK = 128

NP = 10240
ET = R * E
NCHUNK = -(-ET // (NW * CHUNK))
EPW = NCHUNK * CHUNK
ETP = NW * EPW
RPS = NP // NS


def _make_seg_pass(tab_rows, d):
    mesh = plsc.VectorSubcoreMesh(core_axis_name="c", subcore_axis_name="s")

    @functools.partial(
        pl.kernel,
        out_type=jax.ShapeDtypeStruct((NC * NP, d), jnp.float32),
        mesh=mesh,
        scratch_types=[
            pltpu.VMEM((CHUNK,), jnp.int32),
            pltpu.VMEM((CHUNK,), jnp.int32),
            pltpu.VMEM((CHUNK, d), jnp.float32),
            pltpu.VMEM_SHARED((NP, d), jnp.float32),
            pltpu.SemaphoreType.DMA,
        ],
    )
    def seg_pass(tab, src, dst, out, src_v, dst_v, rows_v, acc, sem):
        cid = lax.axis_index("c")
        sid = lax.axis_index("s")
        wid = sid * NC + cid

        zeros16 = jnp.zeros((16,), jnp.float32)

        def zstore(i, carry):
            for k in range(d // 16):
                rows_v[i, pl.ds(k * 16, 16)] = zeros16
            return carry

        lax.fori_loop(0, CHUNK, zstore, 0)

        def zcopy(i, carry):
            pltpu.sync_copy(
                rows_v, acc.at[pl.ds(sid * RPS + i * CHUNK, CHUNK)])
            return carry

        lax.fori_loop(0, RPS // CHUNK, zcopy, 0)
        plsc.subcore_barrier()

        def body(i, carry):
            base = wid * EPW + i * CHUNK
            pltpu.sync_copy(src.at[pl.ds(base, CHUNK)], src_v)
            pltpu.sync_copy(dst.at[pl.ds(base, CHUNK)], dst_v)
            pltpu.async_copy(tab.at[src_v], rows_v, sem).wait()
            pltpu.sync_copy(rows_v, acc.at[dst_v], add=True)
            return carry

        lax.fori_loop(0, NCHUNK, body, 0)
        plsc.subcore_barrier()

        pltpu.sync_copy(
            acc.at[pl.ds(sid * RPS, RPS)],
            out.at[pl.ds(cid * NP + sid * RPS, RPS)])

    return seg_pass


def _combine_mm_body(p_ref, b_ref, w_ref, y_ref):
    h = jnp.maximum(p_ref[0] + p_ref[1] + b_ref[0][None, :], 0.0)
    for r in range(R):
        y_ref[r] = jnp.dot(h, w_ref[r], preferred_element_type=jnp.float32)


def _combine_mm(part, b, w, d2):
    bn = 512
    return pl.pallas_call(
        _combine_mm_body,
        grid=(NP // bn,),
        in_specs=[
            pl.BlockSpec((NC, bn, H), lambda i: (0, i, 0)),
            pl.BlockSpec((1, H), lambda i: (0, 0)),
            pl.BlockSpec((R, H, d2), lambda i: (0, 0, 0)),
        ],
        out_specs=pl.BlockSpec((R, bn, d2), lambda i: (0, i, 0)),
        out_shape=jax.ShapeDtypeStruct((R, NP, d2), jnp.float32),
    )(part, b.reshape(1, H), w)


def _final_body(p_ref, b_ref, o_ref):
    o_ref[...] = (p_ref[0, :, :OUT] + p_ref[1, :, :OUT]
                  + b_ref[0][None, :])


def _final(part, b):
    bn = 512
    return pl.pallas_call(
        _final_body,
        grid=(NP // bn,),
        in_specs=[
            pl.BlockSpec((NC, bn, H), lambda i: (0, i, 0)),
            pl.BlockSpec((1, OUT), lambda i: (0, 0)),
        ],
        out_specs=pl.BlockSpec((bn, OUT), lambda i: (i, 0)),
        out_shape=jax.ShapeDtypeStruct((NP, OUT), jnp.float32),
    )(part, b.reshape(1, OUT))


def kernel(embed, b0, w1, b1, w2, b2, edge_index_0, edge_index_1,
           edge_index_2):
    ei = [edge_index_0, edge_index_1, edge_index_2]
    pad = ETP - ET
    pad_src = jnp.zeros((pad,), jnp.int32)
    pad_dst = jnp.full((pad,), N, jnp.int32)

    src_a = jnp.concatenate([e[0] for e in ei] + [pad_src])
    src_b = jnp.concatenate(
        [ei[r][0] + r * NP for r in range(R)] + [pad_src])
    dst_p = jnp.concatenate([e[1] for e in ei] + [pad_dst])

    part_a = _make_seg_pass(N, H)(embed, src_a, dst_p)
    y = _combine_mm(part_a.reshape(NC, NP, H), b0, w1, H)

    w2p = jnp.concatenate(
        [w2, jnp.zeros((R, H, H - OUT), jnp.float32)], axis=2)
    part_b = _make_seg_pass(R * NP, H)(y.reshape(R * NP, H), src_b, dst_p)
    z = _combine_mm(part_b.reshape(NC, NP, H), b1, w2p, H)

    part_c = _make_seg_pass(R * NP, H)(
        z.reshape(R * NP, H), src_b, dst_p)
    out = _final(part_c.reshape(NC, NP, H), b2)
    return out[:N]


# speedup vs baseline: 9.2506x; 2.4086x over previous
import functools

import jax
import jax.numpy as jnp
from jax import lax
from jax.experimental import pallas as pl
from jax.experimental.pallas import tpu as pltpu
from jax.experimental.pallas import tpu_sc as plsc

N = 10000
H = 128
OUT = 16
R = 3
E = 200000

NC = 2
NS = 16
NW = NC * NS
CHUNK = 128

NP = 10240
ET = R * E
NCHUNK = 160
GB = 16
NGRP = NCHUNK // GB
EPW = NCHUNK * CHUNK
ETP = NW * EPW
RPS = NP // NS


def _make_seg_pass(tab_rows, d):
    mesh = plsc.VectorSubcoreMesh(core_axis_name="c", subcore_axis_name="s")

    @functools.partial(
        pl.kernel,
        out_type=jax.ShapeDtypeStruct((NC * NP, d), jnp.float32),
        mesh=mesh,
        scratch_types=[
            pltpu.VMEM((2, GB, 1, CHUNK), jnp.int32),
            pltpu.VMEM((2, GB, 1, CHUNK), jnp.int32),
            pltpu.VMEM((CHUNK, d), jnp.float32),
            pltpu.VMEM((CHUNK, d), jnp.float32),
            pltpu.VMEM_SHARED((NP, d), jnp.float32),
            pltpu.SemaphoreType.DMA,
            pltpu.SemaphoreType.DMA,
            pltpu.SemaphoreType.DMA,
            pltpu.SemaphoreType.DMA,
        ],
    )
    def seg_pass(tab, src, dst, out, sblk, dblk, rows0, rows1, acc,
                 ssem, dsem, gsem0, gsem1):
        cid = lax.axis_index("c")
        sid = lax.axis_index("s")
        wid = sid * NC + cid
        cbase = wid * NCHUNK

        def load_group(p, g):
            pltpu.async_copy(
                src.at[pl.ds(cbase + g * GB, GB)], sblk.at[p], ssem)
            pltpu.async_copy(
                dst.at[pl.ds(cbase + g * GB, GB)], dblk.at[p], dsem)

        def wait_group(p, g):
            pltpu.make_async_copy(
                src.at[pl.ds(cbase + g * GB, GB)], sblk.at[p], ssem).wait()
            pltpu.make_async_copy(
                dst.at[pl.ds(cbase + g * GB, GB)], dblk.at[p], dsem).wait()

        def g_start(p, j, buf, sem):
            pltpu.async_copy(tab.at[sblk.at[p, j, 0]], buf, sem)

        def g_wait(p, j, buf, sem):
            pltpu.make_async_copy(tab.at[sblk.at[p, j, 0]], buf, sem).wait()

        def s_add(p, j, buf):
            pltpu.sync_copy(buf, acc.at[dblk.at[p, j, 0]], add=True)

        load_group(0, 0)

        zeros16 = jnp.zeros((16,), jnp.float32)

        def zstore(i, carry):
            for k in range(d // 16):
                rows0[i, pl.ds(k * 16, 16)] = zeros16
            return carry

        lax.fori_loop(0, CHUNK, zstore, 0)

        def zcopy(i, carry):
            pltpu.sync_copy(
                rows0, acc.at[pl.ds(sid * RPS + i * CHUNK, CHUNK)])
            return carry

        lax.fori_loop(0, RPS // CHUNK, zcopy, 0)
        wait_group(0, 0)
        plsc.subcore_barrier()

        g_start(0, 0, rows0, gsem0)

        def body(g, carry):
            p = lax.rem(g, 2)
            pn = lax.rem(g + 1, 2)

            @pl.when(g + 1 < NGRP)
            def _():
                load_group(pn, g + 1)

            for j in range(0, GB, 2):
                g_start(p, j + 1, rows1, gsem1)
                g_wait(p, j, rows0, gsem0)
                s_add(p, j, rows0)
                if j + 2 < GB:
                    g_start(p, j + 2, rows0, gsem0)
                else:
                    @pl.when(g + 1 < NGRP)
                    def _():
                        wait_group(pn, g + 1)
                        g_start(pn, 0, rows0, gsem0)
                g_wait(p, j + 1, rows1, gsem1)
                s_add(p, j + 1, rows1)
            return carry

        lax.fori_loop(0, NGRP, body, 0)
        plsc.subcore_barrier()

        pltpu.sync_copy(
            acc.at[pl.ds(sid * RPS, RPS)],
            out.at[pl.ds(cid * NP + sid * RPS, RPS)])

    return seg_pass


def _combine_mm_body(p_ref, b_ref, w_ref, y_ref):
    h = jnp.maximum(p_ref[0] + p_ref[1] + b_ref[0][None, :], 0.0)
    for r in range(R):
        y_ref[r] = jnp.dot(h, w_ref[r], preferred_element_type=jnp.float32)


def _combine_mm(part, b, w, d2):
    bn = 512
    return pl.pallas_call(
        _combine_mm_body,
        grid=(NP // bn,),
        in_specs=[
            pl.BlockSpec((NC, bn, H), lambda i: (0, i, 0)),
            pl.BlockSpec((1, H), lambda i: (0, 0)),
            pl.BlockSpec((R, H, d2), lambda i: (0, 0, 0)),
        ],
        out_specs=pl.BlockSpec((R, bn, d2), lambda i: (0, i, 0)),
        out_shape=jax.ShapeDtypeStruct((R, NP, d2), jnp.float32),
    )(part, b.reshape(1, H), w)


def _final_body(p_ref, b_ref, o_ref):
    o_ref[...] = (p_ref[0, :, :OUT] + p_ref[1, :, :OUT]
                  + b_ref[0][None, :])


def _final(part, b):
    bn = 512
    return pl.pallas_call(
        _final_body,
        grid=(NP // bn,),
        in_specs=[
            pl.BlockSpec((NC, bn, H), lambda i: (0, i, 0)),
            pl.BlockSpec((1, OUT), lambda i: (0, 0)),
        ],
        out_specs=pl.BlockSpec((bn, OUT), lambda i: (i, 0)),
        out_shape=jax.ShapeDtypeStruct((NP, OUT), jnp.float32),
    )(part, b.reshape(1, OUT))


def kernel(embed, b0, w1, b1, w2, b2, edge_index_0, edge_index_1,
           edge_index_2):
    ei = [edge_index_0, edge_index_1, edge_index_2]
    pad = ETP - ET
    pad_src = jnp.arange(pad, dtype=jnp.int32) % N
    pad_dst = N + jnp.arange(pad, dtype=jnp.int32) % (NP - N)

    ishape = (NW * NCHUNK, 1, CHUNK)
    src_a = jnp.concatenate([e[0] for e in ei] + [pad_src]).reshape(ishape)
    src_b = jnp.concatenate(
        [ei[r][0] + r * NP for r in range(R)] + [pad_src]).reshape(ishape)
    dst_p = jnp.concatenate([e[1] for e in ei] + [pad_dst]).reshape(ishape)

    part_a = _make_seg_pass(N, H)(embed, src_a, dst_p)
    y = _combine_mm(part_a.reshape(NC, NP, H), b0, w1, H)

    w2p = jnp.concatenate(
        [w2, jnp.zeros((R, H, H - OUT), jnp.float32)], axis=2)
    part_b = _make_seg_pass(R * NP, H)(y.reshape(R * NP, H), src_b, dst_p)
    z = _combine_mm(part_b.reshape(NC, NP, H), b1, w2p, H)

    part_c = _make_seg_pass(R * NP, H)(
        z.reshape(R * NP, H), src_b, dst_p)
    out = _final(part_c.reshape(NC, NP, H), b2)
    return out[:N]


# speedup vs baseline: 10.2487x; 1.1079x over previous
import functools

import jax
import jax.numpy as jnp
from jax import lax
from jax.experimental import pallas as pl
from jax.experimental.pallas import tpu as pltpu
from jax.experimental.pallas import tpu_sc as plsc

N = 10000
H = 128
OUT = 16
R = 3
E = 200000

NC = 2
NS = 16
NW = NC * NS
CHUNK = 96

NP = 10240
ET = R * E
NCHUNK = 204
GB = 12
NGRP = NCHUNK // GB
EPW = NCHUNK * CHUNK
ETP = NW * EPW
RPS = NP // NS


def _make_seg_pass(tab_rows, d):
    mesh = plsc.VectorSubcoreMesh(core_axis_name="c", subcore_axis_name="s")

    @functools.partial(
        pl.kernel,
        out_type=jax.ShapeDtypeStruct((NC * NP, d), jnp.float32),
        mesh=mesh,
        scratch_types=[
            pltpu.VMEM((2, GB, 1, CHUNK), jnp.int32),
            pltpu.VMEM((2, GB, 1, CHUNK), jnp.int32),
            pltpu.VMEM((3, CHUNK, d), jnp.float32),
            pltpu.VMEM_SHARED((NP, d), jnp.float32),
            pltpu.SemaphoreType.DMA((2,)),
            pltpu.SemaphoreType.DMA((3,)),
            pltpu.SemaphoreType.DMA((3,)),
        ],
    )
    def seg_pass(tab, src, dst, out, sblk, dblk, rows, acc,
                 isem, gsem, ssem):
        cid = lax.axis_index("c")
        sid = lax.axis_index("s")
        wid = sid * NC + cid
        cbase = wid * NCHUNK

        def load_group(p, g):
            pltpu.async_copy(
                src.at[pl.ds(cbase + g * GB, GB)], sblk.at[p], isem.at[0])
            pltpu.async_copy(
                dst.at[pl.ds(cbase + g * GB, GB)], dblk.at[p], isem.at[1])

        def wait_group(p, g):
            pltpu.make_async_copy(
                src.at[pl.ds(cbase + g * GB, GB)], sblk.at[p],
                isem.at[0]).wait()
            pltpu.make_async_copy(
                dst.at[pl.ds(cbase + g * GB, GB)], dblk.at[p],
                isem.at[1]).wait()

        def g_start(p, j, b):
            pltpu.async_copy(tab.at[sblk.at[p, j, 0]], rows.at[b],
                             gsem.at[b])

        def g_wait(p, j, b):
            pltpu.make_async_copy(tab.at[sblk.at[p, j, 0]], rows.at[b],
                                  gsem.at[b]).wait()

        def s_start(p, j, b):
            pltpu.async_copy(rows.at[b], acc.at[dblk.at[p, j, 0]],
                             ssem.at[b], add=True)

        def s_wait(p, j, b):
            pltpu.make_async_copy(rows.at[b], acc.at[dblk.at[p, j, 0]],
                                  ssem.at[b]).wait()

        load_group(0, 0)

        zeros16 = jnp.zeros((16,), jnp.float32)

        def zstore(i, carry):
            for k in range(d // 16):
                rows[0, i, pl.ds(k * 16, 16)] = zeros16
            return carry

        lax.fori_loop(0, CHUNK, zstore, 0)
        nfull = RPS // CHUNK
        nrem = RPS - nfull * CHUNK

        def zcopy(i, carry):
            pltpu.sync_copy(
                rows.at[0], acc.at[pl.ds(sid * RPS + i * CHUNK, CHUNK)])
            return carry

        lax.fori_loop(0, nfull, zcopy, 0)
        if nrem:
            pltpu.sync_copy(
                rows.at[0, pl.ds(0, nrem)],
                acc.at[pl.ds(sid * RPS + nfull * CHUNK, nrem)])
        wait_group(0, 0)
        plsc.subcore_barrier()

        g_start(0, 0, 0)
        g_start(0, 1, 1)

        def group_body(g, p, pn, first, last):
            for j in range(GB):
                b = j % 3
                g_wait(p, j, b)
                s_start(p, j, b)
                if j == 0:
                    if not first:
                        s_wait(pn, GB - 1, 2)
                    if not last:
                        load_group(pn, g + 1)
                else:
                    s_wait(p, j - 1, (j + 2) % 3)
                if j == GB - 2 and not last:
                    wait_group(pn, g + 1)
                if j < GB - 2:
                    g_start(p, j + 2, (j + 2) % 3)
                elif not last:
                    g_start(pn, j + 2 - GB, (j + 2) % 3)

        group_body(0, 0, 1, True, False)

        def body(g, carry):
            p = lax.rem(g, 2)
            pn = lax.rem(g + 1, 2)

            @pl.when(g < NGRP - 1)
            def _():
                group_body(g, p, pn, False, False)

            @pl.when(g == NGRP - 1)
            def _():
                group_body(g, p, pn, False, True)

            return carry

        lax.fori_loop(1, NGRP, body, 0)
        s_wait((NGRP - 1) % 2, GB - 1, (NCHUNK - 1) % 3)
        plsc.subcore_barrier()

        pltpu.sync_copy(
            acc.at[pl.ds(sid * RPS, RPS)],
            out.at[pl.ds(cid * NP + sid * RPS, RPS)])

    return seg_pass


def _combine_mm_body(p_ref, b_ref, w_ref, y_ref):
    h = jnp.maximum(p_ref[0] + p_ref[1] + b_ref[0][None, :], 0.0)
    for r in range(R):
        y_ref[r] = jnp.dot(h, w_ref[r], preferred_element_type=jnp.float32)


def _combine_mm(part, b, w, d2):
    bn = 512
    return pl.pallas_call(
        _combine_mm_body,
        grid=(NP // bn,),
        in_specs=[
            pl.BlockSpec((NC, bn, H), lambda i: (0, i, 0)),
            pl.BlockSpec((1, H), lambda i: (0, 0)),
            pl.BlockSpec((R, H, d2), lambda i: (0, 0, 0)),
        ],
        out_specs=pl.BlockSpec((R, bn, d2), lambda i: (0, i, 0)),
        out_shape=jax.ShapeDtypeStruct((R, NP, d2), jnp.float32),
    )(part, b.reshape(1, H), w)


def _final_body(p_ref, b_ref, o_ref):
    o_ref[...] = (p_ref[0, :, :OUT] + p_ref[1, :, :OUT]
                  + b_ref[0][None, :])


def _final(part, b):
    bn = 512
    return pl.pallas_call(
        _final_body,
        grid=(NP // bn,),
        in_specs=[
            pl.BlockSpec((NC, bn, H), lambda i: (0, i, 0)),
            pl.BlockSpec((1, OUT), lambda i: (0, 0)),
        ],
        out_specs=pl.BlockSpec((bn, OUT), lambda i: (i, 0)),
        out_shape=jax.ShapeDtypeStruct((NP, OUT), jnp.float32),
    )(part, b.reshape(1, OUT))


def kernel(embed, b0, w1, b1, w2, b2, edge_index_0, edge_index_1,
           edge_index_2):
    ei = [edge_index_0, edge_index_1, edge_index_2]
    pad = ETP - ET
    pad_src = jnp.arange(pad, dtype=jnp.int32) % N
    pad_dst = N + jnp.arange(pad, dtype=jnp.int32) % (NP - N)

    ishape = (NW * NCHUNK, 1, CHUNK)
    src_a = jnp.concatenate([e[0] for e in ei] + [pad_src]).reshape(ishape)
    src_b = jnp.concatenate(
        [ei[r][0] + r * NP for r in range(R)] + [pad_src]).reshape(ishape)
    dst_p = jnp.concatenate([e[1] for e in ei] + [pad_dst]).reshape(ishape)

    part_a = _make_seg_pass(N, H)(embed, src_a, dst_p)
    y = _combine_mm(part_a.reshape(NC, NP, H), b0, w1, H)

    w2p = jnp.concatenate(
        [w2, jnp.zeros((R, H, H - OUT), jnp.float32)], axis=2)
    part_b = _make_seg_pass(R * NP, H)(y.reshape(R * NP, H), src_b, dst_p)
    z = _combine_mm(part_b.reshape(NC, NP, H), b1, w2p, H)

    part_c = _make_seg_pass(R * NP, H)(
        z.reshape(R * NP, H), src_b, dst_p)
    out = _final(part_c.reshape(NC, NP, H), b2)
    return out[:N]
